# baseline (speedup 1.0000x reference)
import functools

import jax
import jax.numpy as jnp
from jax import lax
from jax.experimental import pallas as pl
from jax.experimental.pallas import tpu as pltpu
from jax.experimental.pallas import tpu_sc as plsc

N = 10000
E = 320000
C = 128
G = 128

NC = 2
NS = 16
NW = NC * NS
EPT = E // NW
EB = 64
NB = 160
NQ = 8
QB = NB // NQ
PADT = NB * EB - EPT
NP = 10240
RPT = NP // NS
ZC = 128
NZ = RPT // ZC

@functools.cache
def _mesh():
    return plsc.VectorSubcoreMesh(core_axis_name="c", subcore_axis_name="s",
                                  num_cores=NC, num_subcores=NS)


def _deg_body(dst3_hbm, out_hbm, deg_sp, zbuf, ones_v, didx2):
    c = lax.axis_index("c")
    s = lax.axis_index("s")
    wid = c * NS + s

    def _zrow(i, _):
        zbuf[pl.ds(i * 16, 16)] = jnp.zeros((16,), jnp.float32)
        return 0
    lax.fori_loop(0, RPT // 16, _zrow, 0)

    def _orow(i, _):
        ones_v[pl.ds(i * 16, 16)] = jnp.ones((16,), jnp.float32)
        return 0
    lax.fori_loop(0, EB // 16, _orow, 0)

    pltpu.sync_copy(zbuf, deg_sp.at[pl.ds(s * RPT, RPT)])
    pltpu.sync_copy(dst3_hbm.at[wid], didx2)
    plsc.subcore_barrier()

    def _step(j, _):
        pltpu.sync_copy(ones_v, deg_sp.at[didx2.at[j]], add=True)
        return 0
    lax.fori_loop(0, NB, _step, 0)

    plsc.subcore_barrier()
    pltpu.sync_copy(deg_sp.at[pl.ds(s * RPT, RPT)], zbuf)
    pltpu.sync_copy(zbuf, out_hbm.at[pl.ds(c * NP + s * RPT, RPT)])


@functools.cache
def _deg_call():
    return pl.kernel(
        _deg_body,
        out_type=jax.ShapeDtypeStruct((NC * NP,), jnp.float32),
        mesh=_mesh(),
        scratch_types=[
            pltpu.VMEM_SHARED((NP,), jnp.float32),
            pltpu.VMEM((RPT,), jnp.float32),
            pltpu.VMEM((EB,), jnp.float32),
            pltpu.VMEM((NB, EB), jnp.int32),
        ],
    )


def _scatter_body(p_hbm, pk_hbm, out_hbm,
                  acc_sp, qb0, qb1, rows0, rows1, rows2, rows3,
                  g0, g1, g2, g3, s0, s1, s2, s3, q0sem, q1sem):
    c = lax.axis_index("c")
    s = lax.axis_index("s")
    wid = c * NS + s
    rows = (rows0, rows1, rows2, rows3)
    gsem = (g0, g1, g2, g3)
    ssem = (s0, s1, s2, s3)

    def _zrow(r, _):
        for k in range(C // 16):
            rows0[r, pl.ds(k * 16, 16)] = jnp.zeros((16,), jnp.float32)
        return 0
    lax.fori_loop(0, EB, _zrow, 0)

    for j in range(RPT // EB):
        pltpu.sync_copy(rows0, acc_sp.at[pl.ds(s * RPT + j * EB, EB)])
    plsc.subcore_barrier()

    def _gstart(idx_row, b):
        pltpu.async_copy(p_hbm.at[idx_row], rows[b], gsem[b])

    def _gwait(idx_row, b):
        pltpu.make_async_copy(p_hbm.at[idx_row], rows[b], gsem[b]).wait()

    def _sstart(idx_row, b):
        pltpu.async_copy(rows[b], acc_sp.at[idx_row], ssem[b], add=True)

    def _swait(idx_row, b):
        pltpu.make_async_copy(rows[b], acc_sp.at[idx_row], ssem[b]).wait()

    pltpu.sync_copy(pk_hbm.at[wid].at[0], qb0)
    pltpu.async_copy(pk_hbm.at[wid].at[1], qb1, q1sem)

    qbufs = (qb0, qb1)
    qsems = (q0sem, q1sem)
    for q in range(NQ):
        qb = qbufs[q % 2]
        if q > 0:
            pltpu.make_async_copy(pk_hbm.at[wid].at[q], qb,
                                  qsems[q % 2]).wait()
        if q > 0 and q < NQ - 1:
            pltpu.async_copy(pk_hbm.at[wid].at[q + 1], qbufs[(q + 1) % 2],
                             qsems[(q + 1) % 2])

        _gstart(qb.at[0, 0], 0)
        _gstart(qb.at[1, 0], 1)

        def _quad(jj, _):
            for b in range(4):
                k = jj * 4 + b
                bn = (b + 2) % 4
                if b < 2:
                    @pl.when(jj > 0)
                    def _():
                        _swait(qb.at[k - 2, 1], bn)
                    _gstart(qb.at[k + 2, 0], bn)
                else:
                    _swait(qb.at[k - 2, 1], bn)

                    @pl.when(jj < QB // 4 - 1)
                    def _():
                        _gstart(qb.at[k + 2, 0], bn)
                _gwait(qb.at[k, 0], b)
                _sstart(qb.at[k, 1], b)
            return 0
        lax.fori_loop(0, QB // 4, _quad, 0)
        _swait(qb.at[QB - 2, 1], 2)
        _swait(qb.at[QB - 1, 1], 3)

    plsc.subcore_barrier()
    for j in range(RPT // EB):
        pltpu.sync_copy(acc_sp.at[pl.ds(s * RPT + j * EB, EB)], rows0)
        pltpu.sync_copy(rows0,
                        out_hbm.at[c].at[pl.ds(s * RPT + j * EB, EB)])


@functools.cache
def _scatter_call():
    return pl.kernel(
        _scatter_body,
        out_type=jax.ShapeDtypeStruct((NC, NP, C), jnp.float32),
        mesh=_mesh(),
        scratch_types=[
            pltpu.VMEM_SHARED((NP, C), jnp.float32),
            pltpu.VMEM((QB, 2, EB), jnp.int32),
            pltpu.VMEM((QB, 2, EB), jnp.int32),
            pltpu.VMEM((EB, C), jnp.float32),
            pltpu.VMEM((EB, C), jnp.float32),
            pltpu.VMEM((EB, C), jnp.float32),
            pltpu.VMEM((EB, C), jnp.float32),
            pltpu.SemaphoreType.DMA,
            pltpu.SemaphoreType.DMA,
            pltpu.SemaphoreType.DMA,
            pltpu.SemaphoreType.DMA,
            pltpu.SemaphoreType.DMA,
            pltpu.SemaphoreType.DMA,
            pltpu.SemaphoreType.DMA,
            pltpu.SemaphoreType.DMA,
            pltpu.SemaphoreType.DMA,
            pltpu.SemaphoreType.DMA,
        ],
    )


_RB = 1000
_NRB = N // _RB


def _tc_mm_body(x_ref, w_ref, h_ref):
    h_ref[...] = jnp.dot(x_ref[...], w_ref[...],
                         preferred_element_type=jnp.float32)


def _tc_mm(x, w):
    return pl.pallas_call(
        _tc_mm_body,
        grid=(_NRB,),
        in_specs=[
            pl.BlockSpec((_RB, C), lambda i: (i, 0)),
            pl.BlockSpec((C, C), lambda i: (0, 0)),
        ],
        out_specs=pl.BlockSpec((_RB, C), lambda i: (i, 0)),
        out_shape=jax.ShapeDtypeStruct((N, C), jnp.float32),
    )(x, w)


def _tc_first_body(d0_ref, d1_ref, h_ref, p_ref, dis_ref):
    deg = d0_ref[0] + d1_ref[0] + 1.0
    dis = lax.rsqrt(deg)
    p_ref[...] = h_ref[...] * dis
    dis_ref[...] = dis


def _tc_first(degp3, h):
    return pl.pallas_call(
        _tc_first_body,
        grid=(_NRB,),
        in_specs=[
            pl.BlockSpec((1, _RB, 1), lambda i: (0, i, 0)),
            pl.BlockSpec((1, _RB, 1), lambda i: (1, i, 0)),
            pl.BlockSpec((_RB, C), lambda i: (i, 0)),
        ],
        out_specs=[
            pl.BlockSpec((_RB, C), lambda i: (i, 0)),
            pl.BlockSpec((_RB, 1), lambda i: (i, 0)),
        ],
        out_shape=[
            jax.ShapeDtypeStruct((N, C), jnp.float32),
            jax.ShapeDtypeStruct((N, 1), jnp.float32),
        ],
    )(degp3, degp3, h)


def _tc_layer_body(a0_ref, a1_ref, p_ref, dis_ref, b_ref, w_ref, out_ref):
    dis = dis_ref[...]
    t = (a0_ref[0] + a1_ref[0] + p_ref[...]) * dis + b_ref[...]
    t = jnp.maximum(t, 0.0)
    out_ref[...] = jnp.dot(
        t, w_ref[...], preferred_element_type=jnp.float32) * dis


def _tc_layer(acc, p, dis, b, w):
    return pl.pallas_call(
        _tc_layer_body,
        grid=(_NRB,),
        in_specs=[
            pl.BlockSpec((1, _RB, C), lambda i: (0, i, 0)),
            pl.BlockSpec((1, _RB, C), lambda i: (1, i, 0)),
            pl.BlockSpec((_RB, C), lambda i: (i, 0)),
            pl.BlockSpec((_RB, 1), lambda i: (i, 0)),
            pl.BlockSpec((1, C), lambda i: (0, 0)),
            pl.BlockSpec((C, C), lambda i: (0, 0)),
        ],
        out_specs=pl.BlockSpec((_RB, C), lambda i: (i, 0)),
        out_shape=jax.ShapeDtypeStruct((N, C), jnp.float32),
    )(acc, acc, p, dis, b, w)


def _tc_final_body(a0_ref, a1_ref, p_ref, dis_ref, b_ref, bat_ref, wl_ref,
                   out_ref, gsum, cnt):
    i = pl.program_id(0)
    out3 = (a0_ref[0] + a1_ref[0] + p_ref[...]) * dis_ref[...] + b_ref[...]
    onehot = (bat_ref[...] ==
              lax.broadcasted_iota(jnp.int32, (_RB, G), 1)).astype(jnp.float32)
    contrib = lax.dot_general(onehot, out3, (((0,), (0,)), ((), ())),
                              preferred_element_type=jnp.float32)
    ccontrib = lax.dot_general(onehot, jnp.ones((_RB, 1), jnp.float32),
                               (((0,), (0,)), ((), ())),
                               preferred_element_type=jnp.float32)

    @pl.when(i == 0)
    def _():
        gsum[...] = contrib
        cnt[...] = ccontrib

    @pl.when(i > 0)
    def _():
        gsum[...] += contrib
        cnt[...] += ccontrib

    @pl.when(i == _NRB - 1)
    def _():
        g = gsum[...] / jnp.maximum(cnt[...], 1.0)
        out_ref[...] = jnp.dot(g, wl_ref[...],
                               preferred_element_type=jnp.float32)


def _tc_final(acc, p, dis, b, bat, wl):
    return pl.pallas_call(
        _tc_final_body,
        grid=(_NRB,),
        in_specs=[
            pl.BlockSpec((1, _RB, C), lambda i: (0, i, 0)),
            pl.BlockSpec((1, _RB, C), lambda i: (1, i, 0)),
            pl.BlockSpec((_RB, C), lambda i: (i, 0)),
            pl.BlockSpec((_RB, 1), lambda i: (i, 0)),
            pl.BlockSpec((1, C), lambda i: (0, 0)),
            pl.BlockSpec((_RB, 1), lambda i: (i, 0)),
            pl.BlockSpec((C, C), lambda i: (0, 0)),
        ],
        out_specs=pl.BlockSpec((G, C), lambda i: (0, 0)),
        out_shape=jax.ShapeDtypeStruct((G, C), jnp.float32),
        scratch_shapes=[
            pltpu.VMEM((G, C), jnp.float32),
            pltpu.VMEM((G, 1), jnp.float32),
        ],
    )(acc, acc, p, dis, b, bat, wl)


def kernel(x, edge_index, batch, W1, b1, W2, b2, W3, b3, Wlin):
    src = edge_index[0].astype(jnp.int32)
    dst = edge_index[1].astype(jnp.int32)
    bat2d = batch.astype(jnp.int32).reshape(N, 1)

    iw = jnp.arange(NW, dtype=jnp.int32)[:, None]
    it = jnp.arange(PADT, dtype=jnp.int32)[None, :]
    pad_src = (it * 37 + iw * 313) % N
    pad_dst = jnp.broadcast_to(N + it, (NW, PADT))
    src3 = jnp.concatenate([src.reshape(NW, EPT), pad_src],
                           axis=1).reshape(NW, NB, EB)
    dst3 = jnp.concatenate([dst.reshape(NW, EPT), pad_dst],
                           axis=1).reshape(NW, NB, EB)
    pk = jnp.stack([src3, dst3], axis=2).reshape(NW, NQ, QB, 2, EB)

    degp = _deg_call()(dst3)
    h1 = _tc_mm(x, W1)

    scatter = _scatter_call()
    p1, dis = _tc_first(degp.reshape(2, NP, 1), h1)
    acc = scatter(p1, pk)
    p2 = _tc_layer(acc, p1, dis, b1.reshape(1, C), W2)
    acc = scatter(p2, pk)
    p3 = _tc_layer(acc, p2, dis, b2.reshape(1, C), W3)
    acc = scatter(p3, pk)
    return _tc_final(acc, p3, dis, b3.reshape(1, C), bat2d, Wlin)


# speedup vs baseline: 1.0791x; 1.0791x over previous
import functools

import jax
import jax.numpy as jnp
from jax import lax
from jax.experimental import pallas as pl
from jax.experimental.pallas import tpu as pltpu
from jax.experimental.pallas import tpu_sc as plsc

N = 10000
E = 320000
C = 128
G = 128

NC = 2
NS = 16
NW = NC * NS
EPT = E // NW
EB = 128
NB = 80
NQ = 4
QB = NB // NQ
PADT = NB * EB - EPT
NP = 10240
RPT = NP // NS
ZC = 128
NZ = RPT // ZC

@functools.cache
def _mesh():
    return plsc.VectorSubcoreMesh(core_axis_name="c", subcore_axis_name="s",
                                  num_cores=NC, num_subcores=NS)


def _deg_body(dst3_hbm, out_hbm, deg_sp, zbuf, ones_v, didx2):
    c = lax.axis_index("c")
    s = lax.axis_index("s")
    wid = c * NS + s

    def _zrow(i, _):
        zbuf[pl.ds(i * 16, 16)] = jnp.zeros((16,), jnp.float32)
        return 0
    lax.fori_loop(0, RPT // 16, _zrow, 0)

    def _orow(i, _):
        ones_v[pl.ds(i * 16, 16)] = jnp.ones((16,), jnp.float32)
        return 0
    lax.fori_loop(0, EB // 16, _orow, 0)

    pltpu.sync_copy(zbuf, deg_sp.at[pl.ds(s * RPT, RPT)])
    pltpu.sync_copy(dst3_hbm.at[wid], didx2)
    plsc.subcore_barrier()

    def _step(j, _):
        pltpu.sync_copy(ones_v, deg_sp.at[didx2.at[j]], add=True)
        return 0
    lax.fori_loop(0, NB, _step, 0)

    plsc.subcore_barrier()
    pltpu.sync_copy(deg_sp.at[pl.ds(s * RPT, RPT)], zbuf)
    pltpu.sync_copy(zbuf, out_hbm.at[pl.ds(c * NP + s * RPT, RPT)])


@functools.cache
def _deg_call():
    return pl.kernel(
        _deg_body,
        out_type=jax.ShapeDtypeStruct((NC * NP,), jnp.float32),
        mesh=_mesh(),
        scratch_types=[
            pltpu.VMEM_SHARED((NP,), jnp.float32),
            pltpu.VMEM((RPT,), jnp.float32),
            pltpu.VMEM((EB,), jnp.float32),
            pltpu.VMEM((NB, EB), jnp.int32),
        ],
    )


def _scatter_body(p_hbm, pk_hbm, out_hbm,
                  acc_sp, qb0, qb1, rows0, rows1,
                  g0, g1, s0, s1, q0sem, q1sem):
    c = lax.axis_index("c")
    s = lax.axis_index("s")
    wid = c * NS + s

    def _zrow(r, _):
        for k in range(C // 16):
            rows0[r, pl.ds(k * 16, 16)] = jnp.zeros((16,), jnp.float32)
        return 0
    lax.fori_loop(0, ZC, _zrow, 0)

    for j in range(NZ):
        pltpu.sync_copy(rows0, acc_sp.at[pl.ds(s * RPT + j * ZC, ZC)])
    plsc.subcore_barrier()

    def _gstart(idx_row, buf, sem):
        pltpu.async_copy(p_hbm.at[idx_row], buf, sem)

    def _gwait(idx_row, buf, sem):
        pltpu.make_async_copy(p_hbm.at[idx_row], buf, sem).wait()

    def _sstart(idx_row, buf, sem):
        pltpu.async_copy(buf, acc_sp.at[idx_row], sem, add=True)

    def _swait(idx_row, buf, sem):
        pltpu.make_async_copy(buf, acc_sp.at[idx_row], sem).wait()

    pltpu.sync_copy(pk_hbm.at[wid].at[0], qb0)
    pltpu.async_copy(pk_hbm.at[wid].at[1], qb1, q1sem)

    qbufs = (qb0, qb1)
    qsems = (q0sem, q1sem)
    for q in range(NQ):
        qb = qbufs[q % 2]
        if q > 0:
            pltpu.make_async_copy(pk_hbm.at[wid].at[q], qb,
                                  qsems[q % 2]).wait()
        if q > 0 and q < NQ - 1:
            pltpu.async_copy(pk_hbm.at[wid].at[q + 1], qbufs[(q + 1) % 2],
                             qsems[(q + 1) % 2])

        _gstart(qb.at[0, 0], rows0, g0)

        def _pair(jj, _):
            j0 = jj * 2

            @pl.when(jj > 0)
            def _():
                _swait(qb.at[j0 - 1, 1], rows1, s1)
            _gstart(qb.at[j0 + 1, 0], rows1, g1)
            _gwait(qb.at[j0, 0], rows0, g0)
            _sstart(qb.at[j0, 1], rows0, s0)

            @pl.when(jj < QB // 2 - 1)
            def _():
                _swait(qb.at[j0, 1], rows0, s0)
                _gstart(qb.at[j0 + 2, 0], rows0, g0)

            _gwait(qb.at[j0 + 1, 0], rows1, g1)
            _sstart(qb.at[j0 + 1, 1], rows1, s1)
            return 0
        lax.fori_loop(0, QB // 2, _pair, 0)
        _swait(qb.at[QB - 2, 1], rows0, s0)
        _swait(qb.at[QB - 1, 1], rows1, s1)

    plsc.subcore_barrier()
    pltpu.sync_copy(acc_sp.at[pl.ds(s * RPT, RPT)],
                    out_hbm.at[c].at[pl.ds(s * RPT, RPT)])


@functools.cache
def _scatter_call():
    return pl.kernel(
        _scatter_body,
        out_type=jax.ShapeDtypeStruct((NC, NP, C), jnp.float32),
        mesh=_mesh(),
        scratch_types=[
            pltpu.VMEM_SHARED((NP, C), jnp.float32),
            pltpu.VMEM((QB, 2, EB), jnp.int32),
            pltpu.VMEM((QB, 2, EB), jnp.int32),
            pltpu.VMEM((EB, C), jnp.float32),
            pltpu.VMEM((EB, C), jnp.float32),
            pltpu.SemaphoreType.DMA,
            pltpu.SemaphoreType.DMA,
            pltpu.SemaphoreType.DMA,
            pltpu.SemaphoreType.DMA,
            pltpu.SemaphoreType.DMA,
            pltpu.SemaphoreType.DMA,
        ],
    )


_RB = 1000
_NRB = N // _RB


def _tc_mm_body(x_ref, w_ref, h_ref):
    h_ref[...] = jnp.dot(x_ref[...], w_ref[...],
                         preferred_element_type=jnp.float32)


def _tc_mm(x, w):
    return pl.pallas_call(
        _tc_mm_body,
        grid=(_NRB,),
        in_specs=[
            pl.BlockSpec((_RB, C), lambda i: (i, 0)),
            pl.BlockSpec((C, C), lambda i: (0, 0)),
        ],
        out_specs=pl.BlockSpec((_RB, C), lambda i: (i, 0)),
        out_shape=jax.ShapeDtypeStruct((N, C), jnp.float32),
    )(x, w)


def _tc_first_body(d0_ref, d1_ref, h_ref, p_ref, dis_ref):
    deg = d0_ref[0] + d1_ref[0] + 1.0
    dis = lax.rsqrt(deg)
    p_ref[...] = h_ref[...] * dis
    dis_ref[...] = dis


def _tc_first(degp3, h):
    return pl.pallas_call(
        _tc_first_body,
        grid=(_NRB,),
        in_specs=[
            pl.BlockSpec((1, _RB, 1), lambda i: (0, i, 0)),
            pl.BlockSpec((1, _RB, 1), lambda i: (1, i, 0)),
            pl.BlockSpec((_RB, C), lambda i: (i, 0)),
        ],
        out_specs=[
            pl.BlockSpec((_RB, C), lambda i: (i, 0)),
            pl.BlockSpec((_RB, 1), lambda i: (i, 0)),
        ],
        out_shape=[
            jax.ShapeDtypeStruct((N, C), jnp.float32),
            jax.ShapeDtypeStruct((N, 1), jnp.float32),
        ],
    )(degp3, degp3, h)


def _tc_layer_body(a0_ref, a1_ref, p_ref, dis_ref, b_ref, w_ref, out_ref):
    dis = dis_ref[...]
    t = (a0_ref[0] + a1_ref[0] + p_ref[...]) * dis + b_ref[...]
    t = jnp.maximum(t, 0.0)
    out_ref[...] = jnp.dot(
        t, w_ref[...], preferred_element_type=jnp.float32) * dis


def _tc_layer(acc, p, dis, b, w):
    return pl.pallas_call(
        _tc_layer_body,
        grid=(_NRB,),
        in_specs=[
            pl.BlockSpec((1, _RB, C), lambda i: (0, i, 0)),
            pl.BlockSpec((1, _RB, C), lambda i: (1, i, 0)),
            pl.BlockSpec((_RB, C), lambda i: (i, 0)),
            pl.BlockSpec((_RB, 1), lambda i: (i, 0)),
            pl.BlockSpec((1, C), lambda i: (0, 0)),
            pl.BlockSpec((C, C), lambda i: (0, 0)),
        ],
        out_specs=pl.BlockSpec((_RB, C), lambda i: (i, 0)),
        out_shape=jax.ShapeDtypeStruct((N, C), jnp.float32),
    )(acc, acc, p, dis, b, w)


def _tc_final_body(a0_ref, a1_ref, p_ref, dis_ref, b_ref, bat_ref, wl_ref,
                   out_ref, gsum, cnt):
    i = pl.program_id(0)
    out3 = (a0_ref[0] + a1_ref[0] + p_ref[...]) * dis_ref[...] + b_ref[...]
    onehot = (bat_ref[...] ==
              lax.broadcasted_iota(jnp.int32, (_RB, G), 1)).astype(jnp.float32)
    contrib = lax.dot_general(onehot, out3, (((0,), (0,)), ((), ())),
                              preferred_element_type=jnp.float32)
    ccontrib = lax.dot_general(onehot, jnp.ones((_RB, 1), jnp.float32),
                               (((0,), (0,)), ((), ())),
                               preferred_element_type=jnp.float32)

    @pl.when(i == 0)
    def _():
        gsum[...] = contrib
        cnt[...] = ccontrib

    @pl.when(i > 0)
    def _():
        gsum[...] += contrib
        cnt[...] += ccontrib

    @pl.when(i == _NRB - 1)
    def _():
        g = gsum[...] / jnp.maximum(cnt[...], 1.0)
        out_ref[...] = jnp.dot(g, wl_ref[...],
                               preferred_element_type=jnp.float32)


def _tc_final(acc, p, dis, b, bat, wl):
    return pl.pallas_call(
        _tc_final_body,
        grid=(_NRB,),
        in_specs=[
            pl.BlockSpec((1, _RB, C), lambda i: (0, i, 0)),
            pl.BlockSpec((1, _RB, C), lambda i: (1, i, 0)),
            pl.BlockSpec((_RB, C), lambda i: (i, 0)),
            pl.BlockSpec((_RB, 1), lambda i: (i, 0)),
            pl.BlockSpec((1, C), lambda i: (0, 0)),
            pl.BlockSpec((_RB, 1), lambda i: (i, 0)),
            pl.BlockSpec((C, C), lambda i: (0, 0)),
        ],
        out_specs=pl.BlockSpec((G, C), lambda i: (0, 0)),
        out_shape=jax.ShapeDtypeStruct((G, C), jnp.float32),
        scratch_shapes=[
            pltpu.VMEM((G, C), jnp.float32),
            pltpu.VMEM((G, 1), jnp.float32),
        ],
    )(acc, acc, p, dis, b, bat, wl)


def kernel(x, edge_index, batch, W1, b1, W2, b2, W3, b3, Wlin):
    src = edge_index[0].astype(jnp.int32)
    dst = edge_index[1].astype(jnp.int32)
    bat2d = batch.astype(jnp.int32).reshape(N, 1)

    iw = jnp.arange(NW, dtype=jnp.int32)[:, None]
    it = jnp.arange(PADT, dtype=jnp.int32)[None, :]
    pad_src = (it * 37 + iw * 313) % N
    pad_dst = jnp.broadcast_to(N + it, (NW, PADT))
    src3 = jnp.concatenate([src.reshape(NW, EPT), pad_src],
                           axis=1).reshape(NW, NB, EB)
    dst3 = jnp.concatenate([dst.reshape(NW, EPT), pad_dst],
                           axis=1).reshape(NW, NB, EB)
    pk = jnp.stack([src3, dst3], axis=2).reshape(NW, NQ, QB, 2, EB)

    degp = _deg_call()(dst3)
    h1 = _tc_mm(x, W1)

    scatter = _scatter_call()
    p1, dis = _tc_first(degp.reshape(2, NP, 1), h1)
    acc = scatter(p1, pk)
    p2 = _tc_layer(acc, p1, dis, b1.reshape(1, C), W2)
    acc = scatter(p2, pk)
    p3 = _tc_layer(acc, p2, dis, b2.reshape(1, C), W3)
    acc = scatter(p3, pk)
    return _tc_final(acc, p3, dis, b3.reshape(1, C), bat2d, Wlin)


# speedup vs baseline: 1.1078x; 1.0266x over previous
import functools

import jax
import jax.numpy as jnp
from jax import lax
from jax.experimental import pallas as pl
from jax.experimental.pallas import tpu as pltpu
from jax.experimental.pallas import tpu_sc as plsc

N = 10000
E = 320000
C = 128
G = 128

NC = 2
NS = 16
NW = NC * NS
EPT = E // NW
EB = 128
NB = 80
NQ = 4
QB = NB // NQ
PADT = NB * EB - EPT
NP = 10240
RPT = NP // NS
ZC = 128
NZ = RPT // ZC

@functools.cache
def _mesh():
    return plsc.VectorSubcoreMesh(core_axis_name="c", subcore_axis_name="s",
                                  num_cores=NC, num_subcores=NS)


def _deg_body(dst3_hbm, out_hbm, deg_sp, zbuf, ones_v, didx2):
    c = lax.axis_index("c")
    s = lax.axis_index("s")
    wid = c * NS + s

    def _zrow(i, _):
        zbuf[pl.ds(i * 16, 16)] = jnp.zeros((16,), jnp.float32)
        return 0
    lax.fori_loop(0, RPT // 16, _zrow, 0)

    def _orow(i, _):
        ones_v[pl.ds(i * 16, 16)] = jnp.ones((16,), jnp.float32)
        return 0
    lax.fori_loop(0, EB // 16, _orow, 0)

    pltpu.sync_copy(zbuf, deg_sp.at[pl.ds(s * RPT, RPT)])
    pltpu.sync_copy(dst3_hbm.at[wid], didx2)
    plsc.subcore_barrier()

    def _step(j, _):
        pltpu.sync_copy(ones_v, deg_sp.at[didx2.at[j]], add=True)
        return 0
    lax.fori_loop(0, NB, _step, 0)

    plsc.subcore_barrier()
    pltpu.sync_copy(deg_sp.at[pl.ds(s * RPT, RPT)], zbuf)
    pltpu.sync_copy(zbuf, out_hbm.at[pl.ds(c * NP + s * RPT, RPT)])


@functools.cache
def _deg_call():
    return pl.kernel(
        _deg_body,
        out_type=jax.ShapeDtypeStruct((NC * NP,), jnp.float32),
        mesh=_mesh(),
        scratch_types=[
            pltpu.VMEM_SHARED((NP,), jnp.float32),
            pltpu.VMEM((RPT,), jnp.float32),
            pltpu.VMEM((EB,), jnp.float32),
            pltpu.VMEM((NB, EB), jnp.int32),
        ],
    )


def _scatter_body(p_hbm, pk_hbm, out_hbm,
                  acc_sp, qb0, qb1, rows0, rows1,
                  g0, g1, s0, s1, q0sem, q1sem):
    c = lax.axis_index("c")
    s = lax.axis_index("s")
    wid = c * NS + s

    def _zrow(r, _):
        for k in range(C // 16):
            rows0[r, pl.ds(k * 16, 16)] = jnp.zeros((16,), jnp.float32)
        return 0
    lax.fori_loop(0, ZC, _zrow, 0)

    for j in range(NZ):
        pltpu.sync_copy(rows0, acc_sp.at[pl.ds(s * RPT + j * ZC, ZC)])
    plsc.subcore_barrier()

    def _gstart(idx_row, buf, sem):
        pltpu.async_copy(p_hbm.at[idx_row], buf, sem)

    def _gwait(idx_row, buf, sem):
        pltpu.make_async_copy(p_hbm.at[idx_row], buf, sem).wait()

    def _sstart(idx_row, buf, sem):
        pltpu.async_copy(buf, acc_sp.at[idx_row], sem, add=True)

    def _swait(idx_row, buf, sem):
        pltpu.make_async_copy(buf, acc_sp.at[idx_row], sem).wait()

    pltpu.sync_copy(pk_hbm.at[wid].at[0], qb0)
    pltpu.async_copy(pk_hbm.at[wid].at[1], qb1, q1sem)

    qbufs = (qb0, qb1)
    qsems = (q0sem, q1sem)
    for q in range(NQ):
        qb = qbufs[q % 2]
        if q > 0:
            pltpu.make_async_copy(pk_hbm.at[wid].at[q], qb,
                                  qsems[q % 2]).wait()
        if q > 0 and q < NQ - 1:
            pltpu.async_copy(pk_hbm.at[wid].at[q + 1], qbufs[(q + 1) % 2],
                             qsems[(q + 1) % 2])

        _gstart(qb.at[0, 0], rows0, g0)

        def _pair(jj, _):
            j0 = jj * 2

            @pl.when(jj > 0)
            def _():
                _swait(qb.at[j0 - 1, 1], rows1, s1)
            _gstart(qb.at[j0 + 1, 0], rows1, g1)
            _gwait(qb.at[j0, 0], rows0, g0)
            _sstart(qb.at[j0, 1], rows0, s0)

            @pl.when(jj < QB // 2 - 1)
            def _():
                _swait(qb.at[j0, 1], rows0, s0)
                _gstart(qb.at[j0 + 2, 0], rows0, g0)

            _gwait(qb.at[j0 + 1, 0], rows1, g1)
            _sstart(qb.at[j0 + 1, 1], rows1, s1)
            return 0
        lax.fori_loop(0, QB // 2, _pair, 0)
        _swait(qb.at[QB - 2, 1], rows0, s0)
        _swait(qb.at[QB - 1, 1], rows1, s1)

    plsc.subcore_barrier()
    pltpu.sync_copy(acc_sp.at[pl.ds(s * RPT, RPT)],
                    out_hbm.at[c].at[pl.ds(s * RPT, RPT)])


@functools.cache
def _scatter_call():
    return pl.kernel(
        _scatter_body,
        out_type=jax.ShapeDtypeStruct((NC, NP, C), jnp.float32),
        mesh=_mesh(),
        scratch_types=[
            pltpu.VMEM_SHARED((NP, C), jnp.float32),
            pltpu.VMEM((QB, 2, EB), jnp.int32),
            pltpu.VMEM((QB, 2, EB), jnp.int32),
            pltpu.VMEM((EB, C), jnp.float32),
            pltpu.VMEM((EB, C), jnp.float32),
            pltpu.SemaphoreType.DMA,
            pltpu.SemaphoreType.DMA,
            pltpu.SemaphoreType.DMA,
            pltpu.SemaphoreType.DMA,
            pltpu.SemaphoreType.DMA,
            pltpu.SemaphoreType.DMA,
        ],
    )


_RB = 2000
_NRB = N // _RB


def _tc_mm_body(x_ref, w_ref, h_ref):
    h_ref[...] = jnp.dot(x_ref[...], w_ref[...],
                         preferred_element_type=jnp.float32)


def _tc_mm(x, w):
    return pl.pallas_call(
        _tc_mm_body,
        grid=(_NRB,),
        in_specs=[
            pl.BlockSpec((_RB, C), lambda i: (i, 0)),
            pl.BlockSpec((C, C), lambda i: (0, 0)),
        ],
        out_specs=pl.BlockSpec((_RB, C), lambda i: (i, 0)),
        out_shape=jax.ShapeDtypeStruct((N, C), jnp.float32),
    )(x, w)


def _tc_first_body(d0_ref, d1_ref, h_ref, p_ref, dis_ref):
    deg = d0_ref[0] + d1_ref[0] + 1.0
    dis = lax.rsqrt(deg)
    p_ref[...] = h_ref[...] * dis
    dis_ref[...] = dis


def _tc_first(degp3, h):
    return pl.pallas_call(
        _tc_first_body,
        grid=(_NRB,),
        in_specs=[
            pl.BlockSpec((1, _RB, 1), lambda i: (0, i, 0)),
            pl.BlockSpec((1, _RB, 1), lambda i: (1, i, 0)),
            pl.BlockSpec((_RB, C), lambda i: (i, 0)),
        ],
        out_specs=[
            pl.BlockSpec((_RB, C), lambda i: (i, 0)),
            pl.BlockSpec((_RB, 1), lambda i: (i, 0)),
        ],
        out_shape=[
            jax.ShapeDtypeStruct((N, C), jnp.float32),
            jax.ShapeDtypeStruct((N, 1), jnp.float32),
        ],
    )(degp3, degp3, h)


def _tc_layer_body(a0_ref, a1_ref, p_ref, dis_ref, b_ref, w_ref, out_ref):
    dis = dis_ref[...]
    t = (a0_ref[0] + a1_ref[0] + p_ref[...]) * dis + b_ref[...]
    t = jnp.maximum(t, 0.0)
    out_ref[...] = jnp.dot(
        t, w_ref[...], preferred_element_type=jnp.float32) * dis


def _tc_layer(acc, p, dis, b, w):
    return pl.pallas_call(
        _tc_layer_body,
        grid=(_NRB,),
        in_specs=[
            pl.BlockSpec((1, _RB, C), lambda i: (0, i, 0)),
            pl.BlockSpec((1, _RB, C), lambda i: (1, i, 0)),
            pl.BlockSpec((_RB, C), lambda i: (i, 0)),
            pl.BlockSpec((_RB, 1), lambda i: (i, 0)),
            pl.BlockSpec((1, C), lambda i: (0, 0)),
            pl.BlockSpec((C, C), lambda i: (0, 0)),
        ],
        out_specs=pl.BlockSpec((_RB, C), lambda i: (i, 0)),
        out_shape=jax.ShapeDtypeStruct((N, C), jnp.float32),
    )(acc, acc, p, dis, b, w)


def _tc_final_body(a0_ref, a1_ref, p_ref, dis_ref, b_ref, bat_ref, wl_ref,
                   out_ref, gsum, cnt):
    i = pl.program_id(0)
    out3 = (a0_ref[0] + a1_ref[0] + p_ref[...]) * dis_ref[...] + b_ref[...]
    onehot = (bat_ref[...] ==
              lax.broadcasted_iota(jnp.int32, (_RB, G), 1)).astype(jnp.float32)
    contrib = lax.dot_general(onehot, out3, (((0,), (0,)), ((), ())),
                              preferred_element_type=jnp.float32)
    ccontrib = lax.dot_general(onehot, jnp.ones((_RB, 1), jnp.float32),
                               (((0,), (0,)), ((), ())),
                               preferred_element_type=jnp.float32)

    @pl.when(i == 0)
    def _():
        gsum[...] = contrib
        cnt[...] = ccontrib

    @pl.when(i > 0)
    def _():
        gsum[...] += contrib
        cnt[...] += ccontrib

    @pl.when(i == _NRB - 1)
    def _():
        g = gsum[...] / jnp.maximum(cnt[...], 1.0)
        out_ref[...] = jnp.dot(g, wl_ref[...],
                               preferred_element_type=jnp.float32)


def _tc_final(acc, p, dis, b, bat, wl):
    return pl.pallas_call(
        _tc_final_body,
        grid=(_NRB,),
        in_specs=[
            pl.BlockSpec((1, _RB, C), lambda i: (0, i, 0)),
            pl.BlockSpec((1, _RB, C), lambda i: (1, i, 0)),
            pl.BlockSpec((_RB, C), lambda i: (i, 0)),
            pl.BlockSpec((_RB, 1), lambda i: (i, 0)),
            pl.BlockSpec((1, C), lambda i: (0, 0)),
            pl.BlockSpec((_RB, 1), lambda i: (i, 0)),
            pl.BlockSpec((C, C), lambda i: (0, 0)),
        ],
        out_specs=pl.BlockSpec((G, C), lambda i: (0, 0)),
        out_shape=jax.ShapeDtypeStruct((G, C), jnp.float32),
        scratch_shapes=[
            pltpu.VMEM((G, C), jnp.float32),
            pltpu.VMEM((G, 1), jnp.float32),
        ],
    )(acc, acc, p, dis, b, bat, wl)


def kernel(x, edge_index, batch, W1, b1, W2, b2, W3, b3, Wlin):
    src = edge_index[0].astype(jnp.int32)
    dst = edge_index[1].astype(jnp.int32)
    bat2d = batch.astype(jnp.int32).reshape(N, 1)

    iw = jnp.arange(NW, dtype=jnp.int32)[:, None]
    it = jnp.arange(PADT, dtype=jnp.int32)[None, :]
    pad_src = (it * 37 + iw * 313) % N
    pad_dst = jnp.broadcast_to(N + it, (NW, PADT))
    src3 = jnp.concatenate([src.reshape(NW, EPT), pad_src],
                           axis=1).reshape(NW, NB, EB)
    dst3 = jnp.concatenate([dst.reshape(NW, EPT), pad_dst],
                           axis=1).reshape(NW, NB, EB)
    pk = jnp.stack([src3, dst3], axis=2).reshape(NW, NQ, QB, 2, EB)

    degp = _deg_call()(dst3)
    h1 = _tc_mm(x, W1)

    scatter = _scatter_call()
    p1, dis = _tc_first(degp.reshape(2, NP, 1), h1)
    acc = scatter(p1, pk)
    p2 = _tc_layer(acc, p1, dis, b1.reshape(1, C), W2)
    acc = scatter(p2, pk)
    p3 = _tc_layer(acc, p2, dis, b2.reshape(1, C), W3)
    acc = scatter(p3, pk)
    return _tc_final(acc, p3, dis, b3.reshape(1, C), bat2d, Wlin)


# speedup vs baseline: 1.1175x; 1.0087x over previous
import functools

import jax
import jax.numpy as jnp
from jax import lax
from jax.experimental import pallas as pl
from jax.experimental.pallas import tpu as pltpu
from jax.experimental.pallas import tpu_sc as plsc

N = 10000
E = 320000
C = 128
G = 128

NC = 2
NS = 16
NW = NC * NS
EPT = E // NW
EB = 128
NB = 80
NQ = 4
QB = NB // NQ
PADT = NB * EB - EPT
NP = 10240
RPT = NP // NS
ZC = 128
NZ = RPT // ZC

@functools.cache
def _mesh():
    return plsc.VectorSubcoreMesh(core_axis_name="c", subcore_axis_name="s",
                                  num_cores=NC, num_subcores=NS)


def _deg_body(dst3_hbm, out_hbm, deg_sp, zbuf, ones_v, didx2):
    c = lax.axis_index("c")
    s = lax.axis_index("s")
    wid = c * NS + s

    def _zrow(i, _):
        zbuf[pl.ds(i * 16, 16)] = jnp.zeros((16,), jnp.float32)
        return 0
    lax.fori_loop(0, RPT // 16, _zrow, 0)

    def _orow(i, _):
        ones_v[pl.ds(i * 16, 16)] = jnp.ones((16,), jnp.float32)
        return 0
    lax.fori_loop(0, EB // 16, _orow, 0)

    pltpu.sync_copy(zbuf, deg_sp.at[pl.ds(s * RPT, RPT)])
    pltpu.sync_copy(dst3_hbm.at[wid], didx2)
    plsc.subcore_barrier()

    def _step(j, _):
        pltpu.sync_copy(ones_v, deg_sp.at[didx2.at[j]], add=True)
        return 0
    lax.fori_loop(0, NB, _step, 0)

    plsc.subcore_barrier()
    pltpu.sync_copy(deg_sp.at[pl.ds(s * RPT, RPT)], zbuf)
    pltpu.sync_copy(zbuf, out_hbm.at[pl.ds(c * NP + s * RPT, RPT)])


@functools.cache
def _deg_call():
    return pl.kernel(
        _deg_body,
        out_type=jax.ShapeDtypeStruct((NC * NP,), jnp.float32),
        mesh=_mesh(),
        scratch_types=[
            pltpu.VMEM_SHARED((NP,), jnp.float32),
            pltpu.VMEM((RPT,), jnp.float32),
            pltpu.VMEM((EB,), jnp.float32),
            pltpu.VMEM((NB, EB), jnp.int32),
        ],
    )


def _scatter_body(p_hbm, pk_hbm, out_hbm,
                  acc_sp, qb0, qb1, rows0, rows1,
                  g0, g1, s0, s1, q0sem, q1sem):
    c = lax.axis_index("c")
    s = lax.axis_index("s")
    wid = c * NS + s

    def _zrow(r, _):
        for k in range(C // 16):
            rows0[r, pl.ds(k * 16, 16)] = jnp.zeros((16,), jnp.float32)
        return 0
    lax.fori_loop(0, ZC, _zrow, 0)

    for j in range(NZ):
        pltpu.sync_copy(rows0, acc_sp.at[pl.ds(s * RPT + j * ZC, ZC)])
    plsc.subcore_barrier()

    def _gstart(idx_row, buf, sem):
        pltpu.async_copy(p_hbm.at[idx_row], buf, sem)

    def _gwait(idx_row, buf, sem):
        pltpu.make_async_copy(p_hbm.at[idx_row], buf, sem).wait()

    def _sstart(idx_row, buf, sem):
        pltpu.async_copy(buf, acc_sp.at[idx_row], sem, add=True)

    def _swait(idx_row, buf, sem):
        pltpu.make_async_copy(buf, acc_sp.at[idx_row], sem).wait()

    pltpu.sync_copy(pk_hbm.at[wid].at[0], qb0)
    pltpu.async_copy(pk_hbm.at[wid].at[1], qb1, q1sem)

    qbufs = (qb0, qb1)
    qsems = (q0sem, q1sem)
    for q in range(NQ):
        qb = qbufs[q % 2]
        if q > 0:
            pltpu.make_async_copy(pk_hbm.at[wid].at[q], qb,
                                  qsems[q % 2]).wait()
        if q > 0 and q < NQ - 1:
            pltpu.async_copy(pk_hbm.at[wid].at[q + 1], qbufs[(q + 1) % 2],
                             qsems[(q + 1) % 2])

        _gstart(qb.at[0, 0], rows0, g0)

        def _pair(jj, _):
            j0 = jj * 2

            @pl.when(jj > 0)
            def _():
                _swait(qb.at[j0 - 1, 1], rows1, s1)
            _gstart(qb.at[j0 + 1, 0], rows1, g1)
            _gwait(qb.at[j0, 0], rows0, g0)
            _sstart(qb.at[j0, 1], rows0, s0)

            @pl.when(jj < QB // 2 - 1)
            def _():
                _swait(qb.at[j0, 1], rows0, s0)
                _gstart(qb.at[j0 + 2, 0], rows0, g0)

            _gwait(qb.at[j0 + 1, 0], rows1, g1)
            _sstart(qb.at[j0 + 1, 1], rows1, s1)
            return 0
        lax.fori_loop(0, QB // 2, _pair, 0)
        _swait(qb.at[QB - 2, 1], rows0, s0)
        _swait(qb.at[QB - 1, 1], rows1, s1)

    plsc.subcore_barrier()
    pltpu.sync_copy(acc_sp.at[pl.ds(s * RPT, RPT)],
                    out_hbm.at[c].at[pl.ds(s * RPT, RPT)])


@functools.cache
def _scatter_call():
    return pl.kernel(
        _scatter_body,
        out_type=jax.ShapeDtypeStruct((NC, NP, C), jnp.float32),
        mesh=_mesh(),
        scratch_types=[
            pltpu.VMEM_SHARED((NP, C), jnp.float32),
            pltpu.VMEM((QB, 2, EB), jnp.int32),
            pltpu.VMEM((QB, 2, EB), jnp.int32),
            pltpu.VMEM((EB, C), jnp.float32),
            pltpu.VMEM((EB, C), jnp.float32),
            pltpu.SemaphoreType.DMA,
            pltpu.SemaphoreType.DMA,
            pltpu.SemaphoreType.DMA,
            pltpu.SemaphoreType.DMA,
            pltpu.SemaphoreType.DMA,
            pltpu.SemaphoreType.DMA,
        ],
    )


_RB = 5000
_NRB = N // _RB


def _tc_mm_body(x_ref, w_ref, h_ref):
    h_ref[...] = jnp.dot(x_ref[...], w_ref[...],
                         preferred_element_type=jnp.float32)


def _tc_mm(x, w):
    return pl.pallas_call(
        _tc_mm_body,
        grid=(_NRB,),
        in_specs=[
            pl.BlockSpec((_RB, C), lambda i: (i, 0)),
            pl.BlockSpec((C, C), lambda i: (0, 0)),
        ],
        out_specs=pl.BlockSpec((_RB, C), lambda i: (i, 0)),
        out_shape=jax.ShapeDtypeStruct((N, C), jnp.float32),
    )(x, w)


def _tc_first_body(d0_ref, d1_ref, h_ref, p_ref, dis_ref):
    deg = d0_ref[0] + d1_ref[0] + 1.0
    dis = lax.rsqrt(deg)
    p_ref[...] = h_ref[...] * dis
    dis_ref[...] = dis


def _tc_first(degp3, h):
    return pl.pallas_call(
        _tc_first_body,
        grid=(_NRB,),
        in_specs=[
            pl.BlockSpec((1, _RB, 1), lambda i: (0, i, 0)),
            pl.BlockSpec((1, _RB, 1), lambda i: (1, i, 0)),
            pl.BlockSpec((_RB, C), lambda i: (i, 0)),
        ],
        out_specs=[
            pl.BlockSpec((_RB, C), lambda i: (i, 0)),
            pl.BlockSpec((_RB, 1), lambda i: (i, 0)),
        ],
        out_shape=[
            jax.ShapeDtypeStruct((N, C), jnp.float32),
            jax.ShapeDtypeStruct((N, 1), jnp.float32),
        ],
    )(degp3, degp3, h)


def _tc_layer_body(a0_ref, a1_ref, p_ref, dis_ref, b_ref, w_ref, out_ref):
    dis = dis_ref[...]
    t = (a0_ref[0] + a1_ref[0] + p_ref[...]) * dis + b_ref[...]
    t = jnp.maximum(t, 0.0)
    out_ref[...] = jnp.dot(
        t, w_ref[...], preferred_element_type=jnp.float32) * dis


def _tc_layer(acc, p, dis, b, w):
    return pl.pallas_call(
        _tc_layer_body,
        grid=(_NRB,),
        in_specs=[
            pl.BlockSpec((1, _RB, C), lambda i: (0, i, 0)),
            pl.BlockSpec((1, _RB, C), lambda i: (1, i, 0)),
            pl.BlockSpec((_RB, C), lambda i: (i, 0)),
            pl.BlockSpec((_RB, 1), lambda i: (i, 0)),
            pl.BlockSpec((1, C), lambda i: (0, 0)),
            pl.BlockSpec((C, C), lambda i: (0, 0)),
        ],
        out_specs=pl.BlockSpec((_RB, C), lambda i: (i, 0)),
        out_shape=jax.ShapeDtypeStruct((N, C), jnp.float32),
    )(acc, acc, p, dis, b, w)


def _tc_final_body(a0_ref, a1_ref, p_ref, dis_ref, b_ref, bat_ref, wl_ref,
                   out_ref, gsum, cnt):
    i = pl.program_id(0)
    out3 = (a0_ref[0] + a1_ref[0] + p_ref[...]) * dis_ref[...] + b_ref[...]
    onehot = (bat_ref[...] ==
              lax.broadcasted_iota(jnp.int32, (_RB, G), 1)).astype(jnp.float32)
    contrib = lax.dot_general(onehot, out3, (((0,), (0,)), ((), ())),
                              preferred_element_type=jnp.float32)
    ccontrib = lax.dot_general(onehot, jnp.ones((_RB, 1), jnp.float32),
                               (((0,), (0,)), ((), ())),
                               preferred_element_type=jnp.float32)

    @pl.when(i == 0)
    def _():
        gsum[...] = contrib
        cnt[...] = ccontrib

    @pl.when(i > 0)
    def _():
        gsum[...] += contrib
        cnt[...] += ccontrib

    @pl.when(i == _NRB - 1)
    def _():
        g = gsum[...] / jnp.maximum(cnt[...], 1.0)
        out_ref[...] = jnp.dot(g, wl_ref[...],
                               preferred_element_type=jnp.float32)


def _tc_final(acc, p, dis, b, bat, wl):
    return pl.pallas_call(
        _tc_final_body,
        grid=(_NRB,),
        in_specs=[
            pl.BlockSpec((1, _RB, C), lambda i: (0, i, 0)),
            pl.BlockSpec((1, _RB, C), lambda i: (1, i, 0)),
            pl.BlockSpec((_RB, C), lambda i: (i, 0)),
            pl.BlockSpec((_RB, 1), lambda i: (i, 0)),
            pl.BlockSpec((1, C), lambda i: (0, 0)),
            pl.BlockSpec((_RB, 1), lambda i: (i, 0)),
            pl.BlockSpec((C, C), lambda i: (0, 0)),
        ],
        out_specs=pl.BlockSpec((G, C), lambda i: (0, 0)),
        out_shape=jax.ShapeDtypeStruct((G, C), jnp.float32),
        scratch_shapes=[
            pltpu.VMEM((G, C), jnp.float32),
            pltpu.VMEM((G, 1), jnp.float32),
        ],
    )(acc, acc, p, dis, b, bat, wl)


def kernel(x, edge_index, batch, W1, b1, W2, b2, W3, b3, Wlin):
    src = edge_index[0].astype(jnp.int32)
    dst = edge_index[1].astype(jnp.int32)
    bat2d = batch.astype(jnp.int32).reshape(N, 1)

    iw = jnp.arange(NW, dtype=jnp.int32)[:, None]
    it = jnp.arange(PADT, dtype=jnp.int32)[None, :]
    pad_src = (it * 37 + iw * 313) % N
    pad_dst = jnp.broadcast_to(N + it, (NW, PADT))
    src3 = jnp.concatenate([src.reshape(NW, EPT), pad_src],
                           axis=1).reshape(NW, NB, EB)
    dst3 = jnp.concatenate([dst.reshape(NW, EPT), pad_dst],
                           axis=1).reshape(NW, NB, EB)
    pk = jnp.stack([src3, dst3], axis=2).reshape(NW, NQ, QB, 2, EB)

    degp = _deg_call()(dst3)
    h1 = _tc_mm(x, W1)

    scatter = _scatter_call()
    p1, dis = _tc_first(degp.reshape(2, NP, 1), h1)
    acc = scatter(p1, pk)
    p2 = _tc_layer(acc, p1, dis, b1.reshape(1, C), W2)
    acc = scatter(p2, pk)
    p3 = _tc_layer(acc, p2, dis, b2.reshape(1, C), W3)
    acc = scatter(p3, pk)
    return _tc_final(acc, p3, dis, b3.reshape(1, C), bat2d, Wlin)
